# baseline (speedup 1.0000x reference)
import functools

import jax
import jax.numpy as jnp
from jax import lax
from jax.experimental import pallas as pl
from jax.experimental.pallas import tpu as pltpu
from jax.experimental.pallas import tpu_sc as plsc

B = 1024
L = 200
WORD_DIM = 128
POS_DIM = 16
OUT_D = WORD_DIM + 2 * POS_DIM

NC = 2
NS = 16
NW = NC * NS
BPW = B // NW
C0 = 128
C1 = L - C0
NSUB = 2 * BPW
NB = 4
NSTEP = NSUB // NB


def _lookup(word_table, pos1_table, pos2_table, wi, p1i, p2i):
    mesh = plsc.VectorSubcoreMesh(
        core_axis_name="c", subcore_axis_name="s", num_cores=NC, num_subcores=NS
    )

    @functools.partial(
        pl.kernel,
        out_type=jax.ShapeDtypeStruct((B, L, OUT_D), jnp.float32),
        mesh=mesh,
        compiler_params=pltpu.CompilerParams(use_tc_tiling_on_sc=False),
        scratch_types=[
            pltpu.VMEM((BPW, L), jnp.int32),
            pltpu.VMEM((BPW, L), jnp.int32),
            pltpu.VMEM((BPW, L), jnp.int32),
            pltpu.VMEM((NB, C0, WORD_DIM), jnp.float32),
            pltpu.VMEM((NB, C0, POS_DIM), jnp.float32),
            pltpu.VMEM((NB, C0, POS_DIM), jnp.float32),
        ]
        + [pltpu.SemaphoreType.DMA] * NB
        + [pltpu.SemaphoreType.DMA] * NB,
    )
    def k(wt, p1t, p2t, wi_h, p1i_h, p2i_h, out,
          widx_v, p1idx_v, p2idx_v, w_v, p1_v, p2_v, *sems):
        sem_g = sems[:NB]
        sem_s = sems[NB:]
        wid = lax.axis_index("s") * NC + lax.axis_index("c")
        b0 = wid * BPW
        pltpu.sync_copy(wi_h.at[pl.ds(b0, BPW)], widx_v)
        pltpu.sync_copy(p1i_h.at[pl.ds(b0, BPW)], p1idx_v)
        pltpu.sync_copy(p2i_h.at[pl.ds(b0, BPW)], p2idx_v)


        def gather_copies(i, slot, b, issue):
            bl = i * 2 + (slot // 2)
            r0 = (slot % 2) * C0
            n = C0 if slot % 2 == 0 else C1
            idx_w = widx_v.at[bl, pl.ds(r0, n)]
            idx_1 = p1idx_v.at[bl, pl.ds(r0, n)]
            idx_2 = p2idx_v.at[bl, pl.ds(r0, n)]
            f = pltpu.async_copy if issue else (
                lambda s, d, m: pltpu.make_async_copy(s, d, m).wait())
            f(wt.at[idx_w], w_v.at[b, pl.ds(0, n)], sem_g[b])
            f(p1t.at[idx_1], p1_v.at[b, pl.ds(0, n)], sem_g[b])
            f(p2t.at[idx_2], p2_v.at[b, pl.ds(0, n)], sem_g[b])

        def scatter_copies(i, slot, b, issue):
            bl = i * 2 + (slot // 2)
            r0 = (slot % 2) * C0
            n = C0 if slot % 2 == 0 else C1
            bb = b0 + bl
            ow = out.at[bb, pl.ds(r0, n), pl.ds(0, WORD_DIM)]
            o1 = out.at[bb, pl.ds(r0, n), pl.ds(WORD_DIM, POS_DIM)]
            o2 = out.at[bb, pl.ds(r0, n), pl.ds(WORD_DIM + POS_DIM, POS_DIM)]
            f = pltpu.async_copy if issue else (
                lambda s, d, m: pltpu.make_async_copy(s, d, m).wait())
            f(w_v.at[b, pl.ds(0, n)], ow, sem_s[b])
            f(p1_v.at[b, pl.ds(0, n)], o1, sem_s[b])
            f(p2_v.at[b, pl.ds(0, n)], o2, sem_s[b])

        for b in range(NB - 1):
            gather_copies(0, b, b, True)

        def step(i, carry):
            for b in range(NB):
                pb = (b - 1) % NB
                if b == 0:
                    @pl.when(i >= 1)
                    def _():
                        scatter_copies(i, b - 1, pb, False)
                else:
                    scatter_copies(i, b - 1, pb, False)

                if b <= 0:
                    gather_copies(i, b + NB - 1, (b + NB - 1) % NB, True)
                else:
                    @pl.when(i * NB + b + NB - 1 < NSUB)
                    def _():
                        gather_copies(i, b + NB - 1, (b + NB - 1) % NB, True)

                gather_copies(i, b, b, False)
                scatter_copies(i, b, b, True)
            return carry

        lax.fori_loop(0, NSTEP, step, 0)
        scatter_copies(NSTEP - 1, NB - 1, NB - 1, False)

    return k(word_table, pos1_table, pos2_table, wi, p1i, p2i)


def kernel(word_table, pos1_table, pos2_table, word, pos1, pos2):
    wi = word.astype(jnp.int32)
    p1i = pos1.astype(jnp.int32)
    p2i = pos2.astype(jnp.int32)
    return _lookup(word_table, pos1_table, pos2_table, wi, p1i, p2i)
